# baseline (speedup 1.0000x reference)
import functools

import jax
import jax.numpy as jnp
from jax import lax
from jax.experimental import pallas as pl
from jax.experimental.pallas import tpu as pltpu
from jax.experimental.pallas import tpu_sc as plsc

Z = 64
H = Z // 2
B = 16384
V = 2 ** 20
VL = V // 4
NC = 2
NS = 16
NW = NC * NS
BPW = B // NW

_mesh = plsc.VectorSubcoreMesh(core_axis_name="c", subcore_axis_name="s")


def _pack_table(emb):
    e16 = emb.astype(jnp.bfloat16)
    pair = jnp.stack([e16[:, :H], e16[:, H:]], axis=-1)
    w = lax.bitcast_convert_type(pair, jnp.int32)
    return w.reshape(VL, 128)


@functools.partial(
    pl.kernel,
    mesh=_mesh,
    out_type=jax.ShapeDtypeStruct((B, 128), jnp.int32),
    scratch_types=[
        pltpu.VMEM((BPW,), jnp.int32),
        pltpu.VMEM((BPW, 128), jnp.int32),
        pltpu.SemaphoreType.DMA,
    ],
)
def _sc_gather(idx_hbm, table_hbm, out_hbm, idx_v, rows_v, sem):
    wid = lax.axis_index("s") * NC + lax.axis_index("c")
    pltpu.sync_copy(idx_hbm.at[wid], idx_v)

    def body(g, carry):
        vec = idx_v[pl.ds(g * 16, 16)]
        for l in range(16):
            r = vec[l]
            pltpu.async_copy(
                table_hbm.at[pl.ds(r, 1)],
                rows_v.at[pl.ds(g * 16 + l, 1)],
                sem,
            )
        return carry

    lax.fori_loop(0, BPW // 16, body, None)
    pltpu.make_async_copy(table_hbm.at[pl.ds(0, BPW)], rows_v, sem).wait()
    pltpu.sync_copy(rows_v, out_hbm.at[pl.ds(wid * BPW, BPW)])


BB = 2048


def _mlp_body(g_ref, m_ref, w1a_ref, w1b_ref, b1_ref, wmu_ref, bmu_ref,
              wlv_ref, blv_ref, mu_ref, lv_ref):
    w = lax.bitcast_convert_type(g_ref[...], jnp.uint32)
    m = m_ref[...]
    dn = (((1,), (1,)), ((), ()))
    acc = jnp.zeros((BB, Z), jnp.float32)
    for k in range(4):
        wk = w[:, 32 * k:32 * k + 32]
        ak = lax.bitcast_convert_type(wk << 16, jnp.float32)
        bk = lax.bitcast_convert_type(wk & jnp.uint32(0xFFFF0000), jnp.float32)
        term = lax.dot_general(ak, w1a_ref[...], dn,
                               preferred_element_type=jnp.float32,
                               precision=lax.Precision.HIGHEST)
        term = term + lax.dot_general(bk, w1b_ref[...], dn,
                                      preferred_element_type=jnp.float32,
                                      precision=lax.Precision.HIGHEST)
        acc = acc + m[:, k:k + 1] * term
    h = acc + b1_ref[...]
    h = jnp.where(h >= 0, h, 0.01 * h)
    mu_ref[...] = lax.dot_general(h, wmu_ref[...], dn,
                                  preferred_element_type=jnp.float32,
                                  precision=lax.Precision.HIGHEST) + bmu_ref[...]
    lv_ref[...] = lax.dot_general(h, wlv_ref[...], dn,
                                  preferred_element_type=jnp.float32,
                                  precision=lax.Precision.HIGHEST) + blv_ref[...]


def _mlp(g, m, W1, b1, Wmu, bmu, Wlv, blv):
    hspec = pl.BlockSpec((Z, H), lambda i: (0, 0))
    wspec = pl.BlockSpec((Z, Z), lambda i: (0, 0))
    bspec = pl.BlockSpec((1, Z), lambda i: (0, 0))
    return pl.pallas_call(
        _mlp_body,
        grid=(B // BB,),
        in_specs=[
            pl.BlockSpec((BB, 128), lambda i: (i, 0)),
            pl.BlockSpec((BB, 4), lambda i: (i, 0)),
            hspec, hspec, bspec, wspec, bspec, wspec, bspec,
        ],
        out_specs=[
            pl.BlockSpec((BB, Z), lambda i: (i, 0)),
            pl.BlockSpec((BB, Z), lambda i: (i, 0)),
        ],
        out_shape=[
            jax.ShapeDtypeStruct((B, Z), jnp.float32),
            jax.ShapeDtypeStruct((B, Z), jnp.float32),
        ],
    )(g, m, W1[:, :H], W1[:, H:], b1.reshape(1, Z),
      Wmu, bmu.reshape(1, Z), Wlv, blv.reshape(1, Z))


def kernel(x, emb, W1, b1, Wmu, bmu, Wlv, blv):
    xi = x.astype(jnp.int32)
    table = _pack_table(emb)
    xr = (xi >> 2).reshape(NW, BPW)
    m = jax.nn.one_hot(xi & 3, 4, dtype=jnp.float32)
    g = _sc_gather(xr, table)
    mu, lv = _mlp(g, m, W1, b1, Wmu, bmu, Wlv, blv)
    return (mu, lv)


# speedup vs baseline: 1.4058x; 1.4058x over previous
import functools

import jax
import jax.numpy as jnp
from jax import lax
from jax.experimental import pallas as pl
from jax.experimental.pallas import tpu as pltpu
from jax.experimental.pallas import tpu_sc as plsc

Z = 64
H = Z // 2
B = 16384
V = 2 ** 20
VL = V // 4
NC = 2
NS = 16
NW = NC * NS
BPW = B // NW

_mesh = plsc.VectorSubcoreMesh(core_axis_name="c", subcore_axis_name="s")


def _pack_table(emb):
    return emb.reshape(V // 2, 128)


@functools.partial(
    pl.kernel,
    mesh=_mesh,
    out_type=jax.ShapeDtypeStruct((B, 128), jnp.float32),
    scratch_types=[
        pltpu.VMEM((BPW,), jnp.int32),
        pltpu.VMEM((BPW, 128), jnp.float32),
        pltpu.SemaphoreType.DMA,
    ],
)
def _sc_gather(idx_hbm, table_hbm, out_hbm, idx_v, rows_v, sem):
    wid = lax.axis_index("s") * NC + lax.axis_index("c")
    pltpu.sync_copy(idx_hbm.at[wid], idx_v)

    def body(g, carry):
        vec = idx_v[pl.ds(g * 16, 16)]
        for l in range(16):
            r = vec[l]
            pltpu.async_copy(
                table_hbm.at[pl.ds(r, 1)],
                rows_v.at[pl.ds(g * 16 + l, 1)],
                sem,
            )
        return carry

    lax.fori_loop(0, BPW // 16, body, None)
    pltpu.make_async_copy(table_hbm.at[pl.ds(0, BPW)], rows_v, sem).wait()
    pltpu.sync_copy(rows_v, out_hbm.at[pl.ds(wid * BPW, BPW)])


BB = 2048


def _mlp_body(g_ref, m_ref, w1_ref, b1_ref, wmu_ref, bmu_ref,
              wlv_ref, blv_ref, mu_ref, lv_ref):
    w = g_ref[...]
    m = m_ref[...]
    dn = (((1,), (1,)), ((), ()))
    acc = jnp.zeros((BB, Z), jnp.float32)
    for k in range(2):
        gk = w[:, Z * k:Z * k + Z]
        term = lax.dot_general(gk, w1_ref[...], dn,
                               preferred_element_type=jnp.float32,
                               precision=lax.Precision.HIGHEST)
        acc = acc + m[:, k:k + 1] * term
    h = acc + b1_ref[...]
    h = jnp.where(h >= 0, h, 0.01 * h)
    mu_ref[...] = lax.dot_general(h, wmu_ref[...], dn,
                                  preferred_element_type=jnp.float32,
                                  precision=lax.Precision.HIGHEST) + bmu_ref[...]
    lv_ref[...] = lax.dot_general(h, wlv_ref[...], dn,
                                  preferred_element_type=jnp.float32,
                                  precision=lax.Precision.HIGHEST) + blv_ref[...]


def _mlp(g, m, W1, b1, Wmu, bmu, Wlv, blv):
    wspec = pl.BlockSpec((Z, Z), lambda i: (0, 0))
    bspec = pl.BlockSpec((1, Z), lambda i: (0, 0))
    return pl.pallas_call(
        _mlp_body,
        grid=(B // BB,),
        in_specs=[
            pl.BlockSpec((BB, 128), lambda i: (i, 0)),
            pl.BlockSpec((BB, 2), lambda i: (i, 0)),
            wspec, bspec, wspec, bspec, wspec, bspec,
        ],
        out_specs=[
            pl.BlockSpec((BB, Z), lambda i: (i, 0)),
            pl.BlockSpec((BB, Z), lambda i: (i, 0)),
        ],
        out_shape=[
            jax.ShapeDtypeStruct((B, Z), jnp.float32),
            jax.ShapeDtypeStruct((B, Z), jnp.float32),
        ],
    )(g, m, W1, b1.reshape(1, Z),
      Wmu, bmu.reshape(1, Z), Wlv, blv.reshape(1, Z))


def kernel(x, emb, W1, b1, Wmu, bmu, Wlv, blv):
    xi = x.astype(jnp.int32)
    table = _pack_table(emb)
    xr = (xi >> 1).reshape(NW, BPW)
    m = jax.nn.one_hot(xi & 1, 2, dtype=jnp.float32)
    g = _sc_gather(xr, table)
    mu, lv = _mlp(g, m, W1, b1, Wmu, bmu, Wlv, blv)
    return (mu, lv)


# speedup vs baseline: 2.3056x; 1.6401x over previous
import functools

import jax
import jax.numpy as jnp
from jax import lax
from jax.experimental import pallas as pl
from jax.experimental.pallas import tpu as pltpu
from jax.experimental.pallas import tpu_sc as plsc

Z = 64
B = 16384
V = 2 ** 20
NC = 2
NS = 16
NW = NC * NS
BPW = B // NW

_mesh = plsc.VectorSubcoreMesh(core_axis_name="c", subcore_axis_name="s")


@functools.partial(
    pl.kernel,
    mesh=_mesh,
    out_type=jax.ShapeDtypeStruct((B, Z), jnp.float32),
    scratch_types=[
        pltpu.VMEM((BPW,), jnp.int32),
        pltpu.VMEM((BPW, Z), jnp.float32),
        pltpu.SemaphoreType.DMA,
    ],
)
def _sc_gather(idx_hbm, table_hbm, out_hbm, idx_v, rows_v, sem):
    wid = lax.axis_index("s") * NC + lax.axis_index("c")
    pltpu.sync_copy(idx_hbm.at[wid], idx_v)

    def body(g, carry):
        vec = idx_v[pl.ds(g * 16, 16)]
        for l in range(16):
            r = vec[l]
            pltpu.async_copy(
                table_hbm.at[pl.ds(r, 1)],
                rows_v.at[pl.ds(g * 16 + l, 1)],
                sem,
            )
        return carry

    lax.fori_loop(0, BPW // 16, body, None)
    pltpu.make_async_copy(table_hbm.at[pl.ds(0, BPW)], rows_v, sem).wait()
    pltpu.sync_copy(rows_v, out_hbm.at[pl.ds(wid * BPW, BPW)])


BB = 2048


def _mlp_body(g_ref, w1_ref, b1_ref, wmu_ref, bmu_ref, wlv_ref, blv_ref,
              mu_ref, lv_ref):
    g = g_ref[...]
    dn = (((1,), (1,)), ((), ()))
    h = lax.dot_general(g, w1_ref[...], dn,
                        preferred_element_type=jnp.float32,
                        precision=lax.Precision.HIGHEST)
    h = h + b1_ref[...]
    h = jnp.where(h >= 0, h, 0.01 * h)
    mu_ref[...] = lax.dot_general(h, wmu_ref[...], dn,
                                  preferred_element_type=jnp.float32,
                                  precision=lax.Precision.HIGHEST) + bmu_ref[...]
    lv_ref[...] = lax.dot_general(h, wlv_ref[...], dn,
                                  preferred_element_type=jnp.float32,
                                  precision=lax.Precision.HIGHEST) + blv_ref[...]


def _mlp(g, W1, b1, Wmu, bmu, Wlv, blv):
    wspec = pl.BlockSpec((Z, Z), lambda i: (0, 0))
    bspec = pl.BlockSpec((1, Z), lambda i: (0, 0))
    return pl.pallas_call(
        _mlp_body,
        grid=(B // BB,),
        in_specs=[
            pl.BlockSpec((BB, Z), lambda i: (i, 0)),
            wspec, bspec, wspec, bspec, wspec, bspec,
        ],
        out_specs=[
            pl.BlockSpec((BB, Z), lambda i: (i, 0)),
            pl.BlockSpec((BB, Z), lambda i: (i, 0)),
        ],
        out_shape=[
            jax.ShapeDtypeStruct((B, Z), jnp.float32),
            jax.ShapeDtypeStruct((B, Z), jnp.float32),
        ],
    )(g, W1, b1.reshape(1, Z), Wmu, bmu.reshape(1, Z), Wlv, blv.reshape(1, Z))


def kernel(x, emb, W1, b1, Wmu, bmu, Wlv, blv):
    xr = x.astype(jnp.int32).reshape(NW, BPW)
    g = _sc_gather(xr, emb)
    mu, lv = _mlp(g, W1, b1, Wmu, bmu, Wlv, blv)
    return (mu, lv)
